# Initial kernel scaffold; baseline (speedup 1.0000x reference)
import jax
import jax.numpy as jnp
from jax.experimental import pallas as pl


def kernel(xb, yb, table):
    raise NotImplementedError("write your pallas kernel here")


# speedup vs baseline: 1.6453x; 1.6453x over previous
import functools

import jax
import jax.numpy as jnp
from jax import lax
from jax.experimental import pallas as pl
from jax.experimental.pallas import tpu as pltpu
from jax.experimental.pallas import tpu_sc as plsc


def _lse_body(table_ref, out_ref):
    x = table_ref[...]
    m = jnp.max(x, axis=1, keepdims=True)
    s = jnp.sum(jnp.exp(x - m), axis=1, keepdims=True)
    out_ref[...] = m + jnp.log(s)


def _compute_lse(table):
    v = table.shape[0]
    return pl.pallas_call(
        _lse_body,
        out_shape=jax.ShapeDtypeStruct((v, 1), jnp.float32),
    )(table)


@functools.lru_cache(maxsize=None)
def _make_sc_gather(V, D, B):
    info = plsc.get_sparse_core_info()
    NC, NS, L = info.num_cores, info.num_subcores, info.num_lanes
    NW = NC * NS
    assert B % NW == 0
    bpw = B // NW
    CH = 64
    assert bpw % CH == 0
    NIT = bpw // CH
    mesh = plsc.VectorSubcoreMesh(core_axis_name="c", subcore_axis_name="s")

    @functools.partial(
        pl.kernel,
        mesh=mesh,
        compiler_params=pltpu.CompilerParams(
            use_tc_tiling_on_sc=False, needs_layout_passes=False),
        out_type=[
            jax.ShapeDtypeStruct((B, D), jnp.float32),
            jax.ShapeDtypeStruct((NW, L), jnp.float32),
        ],
        scratch_types=[
            pltpu.VMEM((CH,), jnp.int32),
            pltpu.VMEM((CH,), jnp.int32),
            pltpu.VMEM((CH, D), jnp.float32),
            pltpu.VMEM((V,), jnp.float32),
            pltpu.VMEM((L,), jnp.float32),
            pltpu.SemaphoreType.DMA,
        ],
    )
    def sc_kernel(table_hbm, xb_hbm, yb_hbm, lse_hbm, out_hbm, part_hbm,
                  xidx_v, yidx_v, rows_v, lse_v, acc_v, sem):
        wid = lax.axis_index("s") * NC + lax.axis_index("c")
        pltpu.sync_copy(lse_hbm, lse_v)
        acc_v[...] = jnp.zeros((L,), jnp.float32)

        def body(it, carry):
            base = wid * bpw + it * CH
            pltpu.sync_copy(xb_hbm.at[pl.ds(base, CH)], xidx_v)
            pltpu.sync_copy(yb_hbm.at[pl.ds(base, CH)], yidx_v)
            pltpu.async_copy(table_hbm.at[xidx_v], rows_v, sem).wait()
            pltpu.sync_copy(rows_v, out_hbm.at[pl.ds(base, CH)])
            for kk in range(CH // L):
                sl = pl.ds(kk * L, L)
                rloc = lax.iota(jnp.int32, L) + kk * L
                tv = plsc.load_gather(rows_v, [rloc, yidx_v[sl]])
                lv = plsc.load_gather(lse_v, [xidx_v[sl]])
                acc_v[...] = acc_v[...] + (lv - tv)
            return carry

        lax.fori_loop(0, NIT, body, 0)
        pltpu.sync_copy(acc_v, part_hbm.at[wid])

    return sc_kernel


def kernel(xb, yb, table):
    Bb, Tt = xb.shape
    V, D = table.shape
    N = Bb * Tt
    xbf = xb.reshape(N).astype(jnp.int32)
    ybf = yb.reshape(N).astype(jnp.int32)
    lse = _compute_lse(table).reshape(V)
    logits2, parts = _make_sc_gather(V, D, N)(table, xbf, ybf, lse)
    loss = jnp.sum(parts) / N
    return (logits2, loss)


# speedup vs baseline: 2.2388x; 1.3607x over previous
import functools

import jax
import jax.numpy as jnp
from jax import lax
from jax.experimental import pallas as pl
from jax.experimental.pallas import tpu as pltpu
from jax.experimental.pallas import tpu_sc as plsc


def _lse_body(table_ref, out_ref):
    x = table_ref[...]
    m = jnp.max(x, axis=1, keepdims=True)
    s = jnp.sum(jnp.exp(x - m), axis=1, keepdims=True)
    out_ref[...] = m + jnp.log(s)


def _compute_lse(table):
    v = table.shape[0]
    return pl.pallas_call(
        _lse_body,
        out_shape=jax.ShapeDtypeStruct((v, 1), jnp.float32),
    )(table)


@functools.lru_cache(maxsize=None)
def _make_sc_gather(V, D, B):
    info = plsc.get_sparse_core_info()
    NC, NS, L = info.num_cores, info.num_subcores, info.num_lanes
    NW = NC * NS
    assert B % NW == 0
    bpw = B // NW
    CH = 64
    assert bpw % CH == 0
    NIT = bpw // CH
    NTILE = (D + 127) // 128
    mesh = plsc.VectorSubcoreMesh(core_axis_name="c", subcore_axis_name="s")

    DP = NTILE * 128
    TW = D - 128 * (NTILE - 1)
    tail_segs = list(range(0, TW - L + 1, L))
    if tail_segs[-1] != TW - L:
        tail_segs.append(TW - L)

    @functools.partial(
        pl.kernel,
        mesh=mesh,
        out_type=jax.ShapeDtypeStruct((B, D), jnp.float32),
        scratch_types=[
            pltpu.VMEM((CH,), jnp.int32),
            pltpu.VMEM((CH, DP), jnp.float32),
            pltpu.VMEM((CH, TW), jnp.float32),
            pltpu.SemaphoreType.DMA,
        ],
    )
    def sc_kernel(tab_hbm, xb_hbm, out_hbm, xidx_v, rows_v, tail_v, sem):
        wid = lax.axis_index("s") * NC + lax.axis_index("c")

        def body(it, carry):
            base = wid * bpw + it * CH
            pltpu.sync_copy(xb_hbm.at[pl.ds(base, CH)], xidx_v)
            pltpu.async_copy(tab_hbm.at[xidx_v], rows_v, sem).wait()
            for c in range(NTILE - 1):
                pltpu.sync_copy(
                    rows_v.at[:, pl.ds(c * 128, 128)],
                    out_hbm.at[pl.ds(base, CH), pl.ds(c * 128, 128)])
            tb = 128 * (NTILE - 1)
            for t in range(CH):
                for c in tail_segs:
                    tail_v[t, pl.ds(c, L)] = rows_v[t, pl.ds(tb + c, L)]
            pltpu.sync_copy(
                tail_v, out_hbm.at[pl.ds(base, CH), pl.ds(tb, TW)])
            return carry

        lax.fori_loop(0, NIT, body, 0)

    return sc_kernel


@functools.lru_cache(maxsize=None)
def _make_sc_loss(V, D, B):
    info = plsc.get_sparse_core_info()
    NC, NS, L = info.num_cores, info.num_subcores, info.num_lanes
    NW = NC * NS
    bpw = B // NW
    CH = 128
    assert bpw % CH == 0
    NIT = bpw // CH
    mesh = plsc.VectorSubcoreMesh(core_axis_name="c", subcore_axis_name="s")

    @functools.partial(
        pl.kernel,
        mesh=mesh,
        compiler_params=pltpu.CompilerParams(
            use_tc_tiling_on_sc=False, needs_layout_passes=False),
        out_type=jax.ShapeDtypeStruct((NW * L,), jnp.float32),
        scratch_types=[
            pltpu.VMEM((CH,), jnp.int32),
            pltpu.VMEM((CH,), jnp.int32),
            pltpu.VMEM((CH,), jnp.int32),
            pltpu.VMEM((CH,), jnp.float32),
            pltpu.VMEM((CH,), jnp.float32),
            pltpu.VMEM((L,), jnp.float32),
            pltpu.SemaphoreType.DMA,
        ],
    )
    def sc_loss(tflat_hbm, xb_hbm, yb_hbm, lse_hbm, part_hbm,
                xidx_v, yidx_v, fidx_v, lsev_v, tv_v, acc_v, sem):
        wid = lax.axis_index("s") * NC + lax.axis_index("c")
        acc_v[...] = jnp.zeros((L,), jnp.float32)

        def body(it, carry):
            base = wid * bpw + it * CH
            pltpu.sync_copy(xb_hbm.at[pl.ds(base, CH)], xidx_v)
            pltpu.sync_copy(yb_hbm.at[pl.ds(base, CH)], yidx_v)
            for kk in range(CH // L):
                sl = pl.ds(kk * L, L)
                fidx_v[sl] = xidx_v[sl] * D + yidx_v[sl]
            cp_lse = pltpu.async_copy(lse_hbm.at[xidx_v], lsev_v, sem)
            cp_tv = pltpu.async_copy(tflat_hbm.at[fidx_v], tv_v, sem)
            cp_lse.wait()
            cp_tv.wait()
            for kk in range(CH // L):
                sl = pl.ds(kk * L, L)
                acc_v[...] = acc_v[...] + (lsev_v[sl] - tv_v[sl])
            return carry

        lax.fori_loop(0, NIT, body, 0)
        pltpu.sync_copy(acc_v, part_hbm.at[pl.ds(wid * L, L)])

    return sc_loss


def kernel(xb, yb, table):
    Bb, Tt = xb.shape
    V, D = table.shape
    N = Bb * Tt
    NTILE = (D + 127) // 128
    xbf = xb.reshape(N).astype(jnp.int32)
    ybf = yb.reshape(N).astype(jnp.int32)
    lse = _compute_lse(table).reshape(V)
    tab = jnp.pad(table, ((0, 0), (0, NTILE * 128 - D)))
    logits2 = _make_sc_gather(V, D, N)(tab, xbf)
    parts = _make_sc_loss(V, D, N)(table.reshape(V * D), xbf, ybf, lse)
    loss = jnp.sum(parts) / N
    return (logits2, loss)


# speedup vs baseline: 2.4466x; 1.0929x over previous
import functools

import jax
import jax.numpy as jnp
from jax import lax
from jax.experimental import pallas as pl
from jax.experimental.pallas import tpu as pltpu
from jax.experimental.pallas import tpu_sc as plsc


def _lse_body(table_ref, out_ref):
    x = table_ref[...]
    m = jnp.max(x, axis=1, keepdims=True)
    s = jnp.sum(jnp.exp(x - m), axis=1, keepdims=True)
    out_ref[...] = m + jnp.log(s)


def _compute_lse(table):
    v = table.shape[0]
    return pl.pallas_call(
        _lse_body,
        out_shape=jax.ShapeDtypeStruct((v, 1), jnp.float32),
    )(table)


@functools.lru_cache(maxsize=None)
def _make_sc_gather(V, D, B):
    info = plsc.get_sparse_core_info()
    NC, NS, L = info.num_cores, info.num_subcores, info.num_lanes
    NW = NC * NS
    assert B % NW == 0
    bpw = B // NW
    CH = 32
    assert bpw % CH == 0 and CH % L == 0
    NIT = bpw // CH
    NTILE = (D + 127) // 128
    DP = NTILE * 128
    FULLW = 128 * (NTILE - 1)
    TW = D - FULLW
    tail_segs = list(range(0, TW - L + 1, L))
    if tail_segs[-1] != TW - L:
        tail_segs.append(TW - L)
    mesh = plsc.VectorSubcoreMesh(core_axis_name="c", subcore_axis_name="s")

    @functools.partial(
        pl.kernel,
        mesh=mesh,
        out_type=[
            jax.ShapeDtypeStruct((B, D), jnp.float32),
            jax.ShapeDtypeStruct((NW * L,), jnp.float32),
        ],
        scratch_types=[
            pltpu.VMEM((2, CH), jnp.int32),
            pltpu.VMEM((2, CH), jnp.int32),
            pltpu.VMEM((2, CH), jnp.int32),
            pltpu.VMEM((2 * CH, DP), jnp.float32),
            pltpu.VMEM((CH, TW), jnp.float32),
            pltpu.VMEM((2, CH), jnp.float32),
            pltpu.VMEM((2, CH), jnp.float32),
            pltpu.VMEM((L,), jnp.float32),
            pltpu.SemaphoreType.DMA,
            pltpu.SemaphoreType.DMA,
        ],
    )
    def sc_kernel(tab_hbm, tflat_hbm, xb_hbm, yb_hbm, lse_hbm,
                  out_hbm, part_hbm,
                  xidx_v, yidx_v, fidx_v, rows_v, tail_v, lsev_v, tv_v,
                  acc_v, sem0, sem1):
        wid = lax.axis_index("s") * NC + lax.axis_index("c")
        acc_v[...] = jnp.zeros((L,), jnp.float32)
        sems = (sem0, sem1)

        def fire(slot, chunk_i):
            base = wid * bpw + chunk_i * CH
            pltpu.sync_copy(xb_hbm.at[pl.ds(base, CH)], xidx_v.at[slot])
            pltpu.sync_copy(yb_hbm.at[pl.ds(base, CH)], yidx_v.at[slot])
            for kk in range(CH // L):
                sl = pl.ds(kk * L, L)
                fidx_v[slot, sl] = xidx_v[slot, sl] * D + yidx_v[slot, sl]
            pltpu.async_copy(
                tab_hbm.at[xidx_v.at[slot]],
                rows_v.at[pl.ds(slot * CH, CH)], sems[slot])
            pltpu.async_copy(
                lse_hbm.at[xidx_v.at[slot]], lsev_v.at[slot], sems[slot])
            pltpu.async_copy(
                tflat_hbm.at[fidx_v.at[slot]], tv_v.at[slot], sems[slot])

        def drain(slot):
            pltpu.make_async_copy(
                tab_hbm.at[xidx_v.at[slot]],
                rows_v.at[pl.ds(slot * CH, CH)], sems[slot]).wait()
            pltpu.make_async_copy(
                lse_hbm.at[xidx_v.at[slot]], lsev_v.at[slot],
                sems[slot]).wait()
            pltpu.make_async_copy(
                tflat_hbm.at[fidx_v.at[slot]], tv_v.at[slot],
                sems[slot]).wait()

        def consume(slot, chunk_i):
            base = wid * bpw + chunk_i * CH
            rb = slot * CH
            pltpu.sync_copy(
                rows_v.at[pl.ds(rb, CH), pl.ds(0, FULLW)],
                out_hbm.at[pl.ds(base, CH), pl.ds(0, FULLW)])
            for t in range(CH):
                for c in tail_segs:
                    tail_v[t, pl.ds(c, L)] = rows_v[rb + t, pl.ds(FULLW + c, L)]
            pltpu.sync_copy(
                tail_v, out_hbm.at[pl.ds(base, CH), pl.ds(FULLW, TW)])
            for kk in range(CH // L):
                sl = pl.ds(kk * L, L)
                acc_v[...] = acc_v[...] + (lsev_v[slot, sl] - tv_v[slot, sl])

        fire(0, 0)

        def body(i, carry):
            @pl.when(i % 2 == 0)
            def _():
                @pl.when(i + 1 < NIT)
                def _():
                    fire(1, i + 1)
                drain(0)
                consume(0, i)

            @pl.when(i % 2 == 1)
            def _():
                @pl.when(i + 1 < NIT)
                def _():
                    fire(0, i + 1)
                drain(1)
                consume(1, i)

            return carry

        lax.fori_loop(0, NIT, body, 0)
        pltpu.sync_copy(acc_v, part_hbm.at[pl.ds(wid * L, L)])

    return sc_kernel


def kernel(xb, yb, table):
    Bb, Tt = xb.shape
    V, D = table.shape
    N = Bb * Tt
    NTILE = (D + 127) // 128
    xbf = xb.reshape(N).astype(jnp.int32)
    ybf = yb.reshape(N).astype(jnp.int32)
    lse = _compute_lse(table).reshape(V)
    tab = jnp.pad(table, ((0, 0), (0, NTILE * 128 - D)))
    logits2, parts = _make_sc_gather(V, D, N)(
        tab, table.reshape(V * D), xbf, ybf, lse)
    loss = jnp.sum(parts) / N
    return (logits2, loss)
